# baseline (speedup 1.0000x reference)
import jax
import jax.numpy as jnp
from jax import lax
from jax.experimental import pallas as pl
from jax.experimental.pallas import tpu as pltpu
from jax.experimental.pallas import tpu_sc as plsc

N = 10000
E = 160000
D = 256
H = 128
NC = 2
NS = 16
L = 16
K = 32
NCHUNK = 320
EPS = K * NCHUNK
E_PAD = EPS * NS
ECH = E_PAD // K
DCH = ECH // (NC * NS)
G = 16
N_PAD = 10240
RPS = N_PAD // NS
BM = 1000

_f32 = jnp.float32


def _tc_in_body(x_ref, ws_ref, wn_ref, b_ref, s_ref, t_ref):
    x = x_ref[...]
    s_ref[...] = jnp.dot(x, ws_ref[...], preferred_element_type=_f32) + b_ref[...]
    t = jnp.dot(x, wn_ref[...], preferred_element_type=_f32)
    t_ref[0] = t[:, :H]
    t_ref[1] = t[:, H:]


def _tc_in(x, ws, wn, b):
    return pl.pallas_call(
        _tc_in_body,
        grid=(N // BM,),
        in_specs=[
            pl.BlockSpec((BM, D), lambda i: (i, 0)),
            pl.BlockSpec((D, D), lambda i: (0, 0)),
            pl.BlockSpec((D, D), lambda i: (0, 0)),
            pl.BlockSpec((1, D), lambda i: (0, 0)),
        ],
        out_specs=[
            pl.BlockSpec((BM, D), lambda i: (i, 0)),
            pl.BlockSpec((2, BM, H), lambda i: (0, i, 0)),
        ],
        out_shape=[
            jax.ShapeDtypeStruct((N, D), _f32),
            jax.ShapeDtypeStruct((2, N, H), _f32),
        ],
    )(x, ws, wn, b)


def _tc_mid_body(s_ref, a0_ref, a1_ref, d0_ref, d1_ref, ws_ref, wn_ref, b_ref,
                 s2_ref, t_ref):
    deg = d0_ref[...] + d1_ref[...]
    dinv = 1.0 / jnp.maximum(deg, 1.0)
    hn = jnp.concatenate([a0_ref[...], a1_ref[...]], axis=1) * dinv
    h = jnp.maximum(s_ref[...] + hn, 0.0)
    s2_ref[...] = jnp.dot(h, ws_ref[...], preferred_element_type=_f32) + b_ref[...]
    t = jnp.dot(h, wn_ref[...], preferred_element_type=_f32)
    t_ref[0] = t[:, :H]
    t_ref[1] = t[:, H:]


def _tc_mid(s1, a0, a1, d0, d1, ws, wn, b):
    return pl.pallas_call(
        _tc_mid_body,
        grid=(N // BM,),
        in_specs=[
            pl.BlockSpec((BM, D), lambda i: (i, 0)),
            pl.BlockSpec((BM, H), lambda i: (i, 0)),
            pl.BlockSpec((BM, H), lambda i: (i, 0)),
            pl.BlockSpec((BM, 1), lambda i: (i, 0)),
            pl.BlockSpec((BM, 1), lambda i: (i, 0)),
            pl.BlockSpec((D, D), lambda i: (0, 0)),
            pl.BlockSpec((D, D), lambda i: (0, 0)),
            pl.BlockSpec((1, D), lambda i: (0, 0)),
        ],
        out_specs=[
            pl.BlockSpec((BM, D), lambda i: (i, 0)),
            pl.BlockSpec((2, BM, H), lambda i: (0, i, 0)),
        ],
        out_shape=[
            jax.ShapeDtypeStruct((N, D), _f32),
            jax.ShapeDtypeStruct((2, N, H), _f32),
        ],
    )(s1, a0, a1, d0, d1, ws, wn, b)


def _tc_out_body(s_ref, a0_ref, a1_ref, d0_ref, d1_ref, o_ref):
    deg = d0_ref[...] + d1_ref[...]
    dinv = 1.0 / jnp.maximum(deg, 1.0)
    hn = jnp.concatenate([a0_ref[...], a1_ref[...]], axis=1) * dinv
    o_ref[...] = s_ref[...] + hn


def _tc_out(s2, a0, a1, d0, d1):
    return pl.pallas_call(
        _tc_out_body,
        grid=(N // BM,),
        in_specs=[
            pl.BlockSpec((BM, D), lambda i: (i, 0)),
            pl.BlockSpec((BM, H), lambda i: (i, 0)),
            pl.BlockSpec((BM, H), lambda i: (i, 0)),
            pl.BlockSpec((BM, 1), lambda i: (i, 0)),
            pl.BlockSpec((BM, 1), lambda i: (i, 0)),
        ],
        out_specs=pl.BlockSpec((BM, D), lambda i: (i, 0)),
        out_shape=jax.ShapeDtypeStruct((N, D), _f32),
    )(s2, a0, a1, d0, d1)


def _sc_mesh():
    return plsc.VectorSubcoreMesh(core_axis_name="c", subcore_axis_name="s")


def _sc_agg_body(t_hbm, src_hbm, dst_hbm, a_hbm,
                 srcb, dstb, rows0, rows1, rows2, rows3, sem0, sem1, sem2, sem3, acc):
    cid = lax.axis_index("c")
    sid = lax.axis_index("s")
    coff = cid * N

    def zrow(r, _):
        for cc in range(H // L):
            rows0[r, pl.ds(cc * L, L)] = jnp.zeros((L,), _f32)
        return 0
    lax.fori_loop(0, K, zrow, 0)
    for k in range(RPS // K):
        pltpu.sync_copy(rows0, acc.at[pl.ds(sid * RPS + k * K, K)])
    plsc.subcore_barrier()

    def g_start(j, rows, sem):
        pltpu.async_copy(t_hbm.at[srcb.at[j]], rows, sem)

    def g_wait(j, rows, sem):
        pltpu.make_async_copy(t_hbm.at[srcb.at[j]], rows, sem).wait()

    def scat(j, rows):
        pltpu.sync_copy(rows, acc.at[dstb.at[j]], add=True)

    bufs = [(rows0, sem0), (rows1, sem1), (rows2, sem2), (rows3, sem3)]
    DEPTH = len(bufs)

    def group(g, _):
        gbase = sid * NCHUNK + g * G
        pltpu.sync_copy(src_hbm.at[pl.ds(gbase, G)], srcb)
        pltpu.sync_copy(dst_hbm.at[pl.ds(gbase, G)], dstb)

        def offr(j, _):
            for r in range(K // L):
                s = pl.ds(r * L, L)
                srcb[j, s] = srcb[j, s] + coff
            return 0
        lax.fori_loop(0, G, offr, 0)

        for j in range(min(DEPTH - 1, G)):
            g_start(j, *bufs[j % DEPTH])
        for j in range(G):
            if j + DEPTH - 1 < G:
                g_start(j + DEPTH - 1, *bufs[(j + DEPTH - 1) % DEPTH])
            g_wait(j, *bufs[j % DEPTH])
            scat(j, bufs[j % DEPTH][0])
        return 0
    lax.fori_loop(0, NCHUNK // G, group, 0)

    plsc.subcore_barrier()
    lo = sid * RPS
    pltpu.sync_copy(acc.at[pl.ds(lo, RPS)],
                    a_hbm.at[pl.ds(cid * N_PAD + lo, RPS)])


_agg_call_cache = []


def _agg_call(*args):
    if not _agg_call_cache:
        _agg_call_cache.append(pl.kernel(
            _sc_agg_body,
            out_type=jax.ShapeDtypeStruct((2 * N_PAD, H), _f32),
            mesh=_sc_mesh(),
            scratch_types=[
                pltpu.VMEM((G, K), jnp.int32),
                pltpu.VMEM((G, K), jnp.int32),
                pltpu.VMEM((K, H), _f32),
                pltpu.VMEM((K, H), _f32),
                pltpu.VMEM((K, H), _f32),
                pltpu.VMEM((K, H), _f32),
                pltpu.SemaphoreType.DMA,
                pltpu.SemaphoreType.DMA,
                pltpu.SemaphoreType.DMA,
                pltpu.SemaphoreType.DMA,
                pltpu.VMEM_SHARED((N_PAD, H), _f32),
            ],
        ))
    return _agg_call_cache[0](*args)


def _sc_deg_body(dst_hbm, dp_hbm, dstb, ones_v, acc):
    cid = lax.axis_index("c")
    sid = lax.axis_index("s")
    wid = cid * NS + sid

    pltpu.sync_copy(dst_hbm.at[pl.ds(wid * DCH, DCH)], dstb)

    def zrow(r, _):
        for cc in range(H // L):
            ones_v[r, pl.ds(cc * L, L)] = jnp.zeros((L,), _f32)
        return 0
    lax.fori_loop(0, K, zrow, 0)
    for k in range(RPS // K):
        pltpu.sync_copy(ones_v, acc.at[pl.ds(sid * RPS + k * K, K)])

    def orow(r, _):
        for cc in range(H // L):
            ones_v[r, pl.ds(cc * L, L)] = jnp.ones((L,), _f32)
        return 0
    lax.fori_loop(0, K, orow, 0)
    plsc.subcore_barrier()

    def chunk(j, _):
        pltpu.sync_copy(ones_v, acc.at[dstb.at[j]], add=True)
        return 0
    lax.fori_loop(0, DCH, chunk, 0)

    plsc.subcore_barrier()
    lo = sid * RPS
    pltpu.sync_copy(acc.at[pl.ds(lo, RPS)],
                    dp_hbm.at[pl.ds(cid * N_PAD + lo, RPS)])


_deg_call_cache = []


def _deg_call(*args):
    if not _deg_call_cache:
        _deg_call_cache.append(pl.kernel(
            _sc_deg_body,
            out_type=jax.ShapeDtypeStruct((2 * N_PAD, H), _f32),
            mesh=_sc_mesh(),
            scratch_types=[
                pltpu.VMEM((DCH, K), jnp.int32),
                pltpu.VMEM((K, H), _f32),
                pltpu.VMEM_SHARED((N_PAD, H), _f32),
            ],
        ))
    return _deg_call_cache[0](*args)


def kernel(x, edge_index, W_self1, W_neigh1, b1, W_self2, W_neigh2, b2):
    src = edge_index[0].astype(jnp.int32)
    dst = edge_index[1].astype(jnp.int32)
    pad = E_PAD - E
    src2 = jnp.concatenate([src, jnp.zeros((pad,), jnp.int32)]).reshape(ECH, K)
    dst2 = jnp.concatenate([dst, jnp.full((pad,), N, jnp.int32)]).reshape(ECH, K)

    dp = _deg_call(dst2)
    d0 = dp[:N, 0:1]
    d1 = dp[N_PAD:N_PAD + N, 0:1]

    s1, t1 = _tc_in(x, W_self1, W_neigh1, b1.reshape(1, D))
    agg1 = _agg_call(t1.reshape(2 * N, H), src2, dst2)
    s2, t2 = _tc_mid(s1, agg1[:N], agg1[N_PAD:N_PAD + N], d0, d1,
                     W_self2, W_neigh2, b2.reshape(1, D))
    agg2 = _agg_call(t2.reshape(2 * N, H), src2, dst2)
    return _tc_out(s2, agg2[:N], agg2[N_PAD:N_PAD + N], d0, d1)


# speedup vs baseline: 1.1311x; 1.1311x over previous
import jax
import jax.numpy as jnp
from jax import lax
from jax.experimental import pallas as pl
from jax.experimental.pallas import tpu as pltpu
from jax.experimental.pallas import tpu_sc as plsc

N = 10000
E = 160000
D = 256
H = 128
NC = 2
NS = 16
L = 16
K = 80
NCHUNK = 128
EPS = K * NCHUNK
E_PAD = EPS * NS
ECH = E_PAD // K
DCH = ECH // (NC * NS)
G = 16
N_PAD = 10240
RPS = N_PAD // NS
BM = 1000

_f32 = jnp.float32


def _tc_in_body(x_ref, ws_ref, wn_ref, b_ref, s_ref, t_ref):
    x = x_ref[...]
    s_ref[...] = jnp.dot(x, ws_ref[...], preferred_element_type=_f32) + b_ref[...]
    t = jnp.dot(x, wn_ref[...], preferred_element_type=_f32)
    t_ref[0] = t[:, :H]
    t_ref[1] = t[:, H:]


def _tc_in(x, ws, wn, b):
    return pl.pallas_call(
        _tc_in_body,
        grid=(N // BM,),
        in_specs=[
            pl.BlockSpec((BM, D), lambda i: (i, 0)),
            pl.BlockSpec((D, D), lambda i: (0, 0)),
            pl.BlockSpec((D, D), lambda i: (0, 0)),
            pl.BlockSpec((1, D), lambda i: (0, 0)),
        ],
        out_specs=[
            pl.BlockSpec((BM, D), lambda i: (i, 0)),
            pl.BlockSpec((2, BM, H), lambda i: (0, i, 0)),
        ],
        out_shape=[
            jax.ShapeDtypeStruct((N, D), _f32),
            jax.ShapeDtypeStruct((2, N, H), _f32),
        ],
    )(x, ws, wn, b)


def _tc_mid_body(s_ref, a0_ref, a1_ref, d0_ref, d1_ref, ws_ref, wn_ref, b_ref,
                 s2_ref, t_ref):
    deg = d0_ref[...] + d1_ref[...]
    dinv = 1.0 / jnp.maximum(deg, 1.0)
    hn = jnp.concatenate([a0_ref[...], a1_ref[...]], axis=1) * dinv
    h = jnp.maximum(s_ref[...] + hn, 0.0)
    s2_ref[...] = jnp.dot(h, ws_ref[...], preferred_element_type=_f32) + b_ref[...]
    t = jnp.dot(h, wn_ref[...], preferred_element_type=_f32)
    t_ref[0] = t[:, :H]
    t_ref[1] = t[:, H:]


def _tc_mid(s1, a0, a1, d0, d1, ws, wn, b):
    return pl.pallas_call(
        _tc_mid_body,
        grid=(N // BM,),
        in_specs=[
            pl.BlockSpec((BM, D), lambda i: (i, 0)),
            pl.BlockSpec((BM, H), lambda i: (i, 0)),
            pl.BlockSpec((BM, H), lambda i: (i, 0)),
            pl.BlockSpec((BM, 1), lambda i: (i, 0)),
            pl.BlockSpec((BM, 1), lambda i: (i, 0)),
            pl.BlockSpec((D, D), lambda i: (0, 0)),
            pl.BlockSpec((D, D), lambda i: (0, 0)),
            pl.BlockSpec((1, D), lambda i: (0, 0)),
        ],
        out_specs=[
            pl.BlockSpec((BM, D), lambda i: (i, 0)),
            pl.BlockSpec((2, BM, H), lambda i: (0, i, 0)),
        ],
        out_shape=[
            jax.ShapeDtypeStruct((N, D), _f32),
            jax.ShapeDtypeStruct((2, N, H), _f32),
        ],
    )(s1, a0, a1, d0, d1, ws, wn, b)


def _tc_out_body(s_ref, a0_ref, a1_ref, d0_ref, d1_ref, o_ref):
    deg = d0_ref[...] + d1_ref[...]
    dinv = 1.0 / jnp.maximum(deg, 1.0)
    hn = jnp.concatenate([a0_ref[...], a1_ref[...]], axis=1) * dinv
    o_ref[...] = s_ref[...] + hn


def _tc_out(s2, a0, a1, d0, d1):
    return pl.pallas_call(
        _tc_out_body,
        grid=(N // BM,),
        in_specs=[
            pl.BlockSpec((BM, D), lambda i: (i, 0)),
            pl.BlockSpec((BM, H), lambda i: (i, 0)),
            pl.BlockSpec((BM, H), lambda i: (i, 0)),
            pl.BlockSpec((BM, 1), lambda i: (i, 0)),
            pl.BlockSpec((BM, 1), lambda i: (i, 0)),
        ],
        out_specs=pl.BlockSpec((BM, D), lambda i: (i, 0)),
        out_shape=jax.ShapeDtypeStruct((N, D), _f32),
    )(s2, a0, a1, d0, d1)


def _sc_mesh():
    return plsc.VectorSubcoreMesh(core_axis_name="c", subcore_axis_name="s")


def _sc_agg_body(t_hbm, src_hbm, dst_hbm, a_hbm,
                 srcb, dstb, rows0, rows1, sem0, sem1, acc):
    cid = lax.axis_index("c")
    sid = lax.axis_index("s")
    coff = cid * N

    def zrow(r, _):
        for cc in range(H // L):
            rows0[r, pl.ds(cc * L, L)] = jnp.zeros((L,), _f32)
        return 0
    lax.fori_loop(0, K, zrow, 0)
    for k in range(RPS // K):
        pltpu.sync_copy(rows0, acc.at[pl.ds(sid * RPS + k * K, K)])
    plsc.subcore_barrier()

    def g_start(j, rows, sem):
        pltpu.async_copy(t_hbm.at[srcb.at[j]], rows, sem)

    def g_wait(j, rows, sem):
        pltpu.make_async_copy(t_hbm.at[srcb.at[j]], rows, sem).wait()

    def scat(j, rows):
        pltpu.sync_copy(rows, acc.at[dstb.at[j]], add=True)

    bufs = [(rows0, sem0), (rows1, sem1)]
    DEPTH = len(bufs)

    def group(g, _):
        gbase = sid * NCHUNK + g * G
        pltpu.sync_copy(src_hbm.at[pl.ds(gbase, G)], srcb)
        pltpu.sync_copy(dst_hbm.at[pl.ds(gbase, G)], dstb)

        def offr(j, _):
            for r in range(K // L):
                s = pl.ds(r * L, L)
                srcb[j, s] = srcb[j, s] + coff
            return 0
        lax.fori_loop(0, G, offr, 0)

        for j in range(min(DEPTH - 1, G)):
            g_start(j, *bufs[j % DEPTH])
        for j in range(G):
            if j + DEPTH - 1 < G:
                g_start(j + DEPTH - 1, *bufs[(j + DEPTH - 1) % DEPTH])
            g_wait(j, *bufs[j % DEPTH])
            scat(j, bufs[j % DEPTH][0])
        return 0
    lax.fori_loop(0, NCHUNK // G, group, 0)

    plsc.subcore_barrier()
    lo = sid * RPS
    pltpu.sync_copy(acc.at[pl.ds(lo, RPS)],
                    a_hbm.at[pl.ds(cid * N_PAD + lo, RPS)])


_agg_call_cache = []


def _agg_call(*args):
    if not _agg_call_cache:
        _agg_call_cache.append(pl.kernel(
            _sc_agg_body,
            out_type=jax.ShapeDtypeStruct((2 * N_PAD, H), _f32),
            mesh=_sc_mesh(),
            scratch_types=[
                pltpu.VMEM((G, K), jnp.int32),
                pltpu.VMEM((G, K), jnp.int32),
                pltpu.VMEM((K, H), _f32),
                pltpu.VMEM((K, H), _f32),
                pltpu.SemaphoreType.DMA,
                pltpu.SemaphoreType.DMA,
                pltpu.VMEM_SHARED((N_PAD, H), _f32),
            ],
        ))
    return _agg_call_cache[0](*args)


def _sc_deg_body(dst_hbm, dp_hbm, dstb, ones_v, acc):
    cid = lax.axis_index("c")
    sid = lax.axis_index("s")
    wid = cid * NS + sid

    pltpu.sync_copy(dst_hbm.at[pl.ds(wid * DCH, DCH)], dstb)

    def zrow(r, _):
        for cc in range(H // L):
            ones_v[r, pl.ds(cc * L, L)] = jnp.zeros((L,), _f32)
        return 0
    lax.fori_loop(0, K, zrow, 0)
    for k in range(RPS // K):
        pltpu.sync_copy(ones_v, acc.at[pl.ds(sid * RPS + k * K, K)])

    def orow(r, _):
        for cc in range(H // L):
            ones_v[r, pl.ds(cc * L, L)] = jnp.ones((L,), _f32)
        return 0
    lax.fori_loop(0, K, orow, 0)
    plsc.subcore_barrier()

    def chunk(j, _):
        pltpu.sync_copy(ones_v, acc.at[dstb.at[j]], add=True)
        return 0
    lax.fori_loop(0, DCH, chunk, 0)

    plsc.subcore_barrier()
    lo = sid * RPS
    pltpu.sync_copy(acc.at[pl.ds(lo, RPS)],
                    dp_hbm.at[pl.ds(cid * N_PAD + lo, RPS)])


_deg_call_cache = []


def _deg_call(*args):
    if not _deg_call_cache:
        _deg_call_cache.append(pl.kernel(
            _sc_deg_body,
            out_type=jax.ShapeDtypeStruct((2 * N_PAD, H), _f32),
            mesh=_sc_mesh(),
            scratch_types=[
                pltpu.VMEM((DCH, K), jnp.int32),
                pltpu.VMEM((K, H), _f32),
                pltpu.VMEM_SHARED((N_PAD, H), _f32),
            ],
        ))
    return _deg_call_cache[0](*args)


def kernel(x, edge_index, W_self1, W_neigh1, b1, W_self2, W_neigh2, b2):
    src = edge_index[0].astype(jnp.int32)
    dst = edge_index[1].astype(jnp.int32)
    pad = E_PAD - E
    src2 = jnp.concatenate([src, jnp.zeros((pad,), jnp.int32)]).reshape(ECH, K)
    dst2 = jnp.concatenate([dst, jnp.full((pad,), N, jnp.int32)]).reshape(ECH, K)

    dp = _deg_call(dst2)
    d0 = dp[:N, 0:1]
    d1 = dp[N_PAD:N_PAD + N, 0:1]

    s1, t1 = _tc_in(x, W_self1, W_neigh1, b1.reshape(1, D))
    agg1 = _agg_call(t1.reshape(2 * N, H), src2, dst2)
    s2, t2 = _tc_mid(s1, agg1[:N], agg1[N_PAD:N_PAD + N], d0, d1,
                     W_self2, W_neigh2, b2.reshape(1, D))
    agg2 = _agg_call(t2.reshape(2 * N, H), src2, dst2)
    return _tc_out(s2, agg2[:N], agg2[N_PAD:N_PAD + N], d0, d1)
